# baseline (speedup 1.0000x reference)
import functools

import jax
import jax.numpy as jnp
from jax import lax
from jax.experimental import pallas as pl
from jax.experimental.pallas import tpu as pltpu
from jax.experimental.pallas import tpu_sc as plsc

NC = 2
NS = 16
NW = NC * NS
CH = 128
LANES = 16


def _sc_mesh():
    return plsc.VectorSubcoreMesh(
        core_axis_name="c", subcore_axis_name="s",
        num_cores=NC, num_subcores=NS)


def _zero_vmem_2d(ref, rows, cols):
    def fill(r, _):
        for k in range(cols // LANES):
            ref[r, pl.ds(k * LANES, LANES)] = jnp.zeros((LANES,), jnp.float32)
        return 0
    lax.fori_loop(0, rows, fill, 0)


def _zero_vmem_1d(ref, n):
    def fill(r, _):
        ref[pl.ds(r * LANES, LANES)] = jnp.zeros((LANES,), jnp.float32)
        return 0
    lax.fori_loop(0, n // LANES, fill, 0)


def _sc_degree(dstp, n_pad, n_chunks):
    rpt = n_pad // NS

    @functools.partial(
        pl.kernel,
        out_type=jax.ShapeDtypeStruct((NC * n_pad,), jnp.float32),
        mesh=_sc_mesh(),
        scratch_types=[
            pltpu.VMEM_SHARED((n_pad,), jnp.float32),
            pltpu.VMEM((n_chunks, CH), jnp.int32),
            pltpu.VMEM((CH,), jnp.float32),
            pltpu.VMEM((rpt,), jnp.float32),
            pltpu.SemaphoreType.DMA,
        ],
    )
    def k(dst_hbm, out_hbm, acc_sh, dst_v, ones_v, buf_v, sem):
        c = lax.axis_index("c")
        s = lax.axis_index("s")
        w = c * NS + s
        pltpu.sync_copy(dst_hbm.at[w], dst_v)

        def fill_ones(r, _):
            ones_v[pl.ds(r * LANES, LANES)] = jnp.ones((LANES,), jnp.float32)
            return 0
        lax.fori_loop(0, CH // LANES, fill_ones, 0)
        _zero_vmem_1d(buf_v, rpt)
        pltpu.sync_copy(buf_v, acc_sh.at[pl.ds(s * rpt, rpt)])
        plsc.subcore_barrier()

        depth = 4

        def fire(j, _):
            @pl.when(j >= depth)
            def _():
                pltpu.make_async_copy(ones_v, acc_sh.at[dst_v.at[j - depth]],
                                      sem).wait()
            pltpu.async_copy(ones_v, acc_sh.at[dst_v.at[j]], sem, add=True)
            return 0
        lax.fori_loop(0, n_chunks, fire, 0)

        def drain(j, _):
            pltpu.make_async_copy(ones_v, acc_sh.at[dst_v.at[j]], sem).wait()
            return 0
        lax.fori_loop(n_chunks - depth, n_chunks, drain, 0)
        plsc.subcore_barrier()
        pltpu.sync_copy(acc_sh.at[pl.ds(s * rpt, rpt)],
                        out_hbm.at[pl.ds(c * n_pad + s * rpt, rpt)])

    return k(dstp)


def _sc_edge_agg(g1, srcp, dstp, n_pad, n_chunks):
    d = g1.shape[1]
    rpt = n_pad // NS

    @functools.partial(
        pl.kernel,
        out_type=jax.ShapeDtypeStruct((NC * n_pad, d), jnp.float32),
        mesh=_sc_mesh(),
        scratch_types=[
            pltpu.VMEM_SHARED((n_pad, d), jnp.float32),
            pltpu.VMEM((n_chunks, CH), jnp.int32),
            pltpu.VMEM((n_chunks, CH), jnp.int32),
            pltpu.VMEM((CH, d), jnp.float32),
            pltpu.SemaphoreType.DMA,
        ],
    )
    def k(g1_hbm, src_hbm, dst_hbm, out_hbm, acc_sh, src_v, dst_v, rows_v,
          sem):
        c = lax.axis_index("c")
        s = lax.axis_index("s")
        w = c * NS + s
        pltpu.sync_copy(src_hbm.at[w], src_v)
        pltpu.sync_copy(dst_hbm.at[w], dst_v)
        _zero_vmem_2d(rows_v, CH, d)
        for i in range(rpt // CH):
            pltpu.sync_copy(rows_v, acc_sh.at[pl.ds(s * rpt + i * CH, CH)])
        plsc.subcore_barrier()

        def chunk(j, _):
            pltpu.async_copy(g1_hbm.at[src_v.at[j]], rows_v, sem).wait()
            pltpu.sync_copy(rows_v, acc_sh.at[dst_v.at[j]], add=True)
            return 0
        lax.fori_loop(0, n_chunks, chunk, 0)
        plsc.subcore_barrier()
        pltpu.sync_copy(acc_sh.at[pl.ds(s * rpt, rpt)],
                        out_hbm.at[pl.ds(c * n_pad + s * rpt, rpt)])

    return k(g1, srcp, dstp)


def _sc_edge_agg_scalar(g2, srcp, dstp, n_pad, n_chunks):
    rpt = n_pad // NS

    nbuf = 2
    assert n_chunks % nbuf == 0

    @functools.partial(
        pl.kernel,
        out_type=jax.ShapeDtypeStruct((NC * n_pad,), jnp.float32),
        mesh=_sc_mesh(),
        scratch_types=[
            pltpu.VMEM_SHARED((n_pad,), jnp.float32),
            pltpu.VMEM_SHARED((n_pad,), jnp.float32),
            pltpu.VMEM((n_chunks, CH), jnp.int32),
            pltpu.VMEM((n_chunks, CH), jnp.int32),
            pltpu.VMEM((nbuf, CH), jnp.float32),
            pltpu.VMEM((rpt,), jnp.float32),
            [pltpu.SemaphoreType.DMA] * nbuf,
            [pltpu.SemaphoreType.DMA] * nbuf,
        ],
    )
    def k(g2_hbm, src_hbm, dst_hbm, out_hbm, acc_sh, tab_sh, src_v, dst_v,
          vals_v, buf_v, gsems, ssems):
        c = lax.axis_index("c")
        s = lax.axis_index("s")
        w = c * NS + s
        pltpu.sync_copy(src_hbm.at[w], src_v)
        pltpu.sync_copy(dst_hbm.at[w], dst_v)
        @pl.when(s == 0)
        def _():
            pltpu.sync_copy(g2_hbm, tab_sh)
        _zero_vmem_1d(buf_v, rpt)
        pltpu.sync_copy(buf_v, acc_sh.at[pl.ds(s * rpt, rpt)])
        plsc.subcore_barrier()

        def gfire(b, j):
            pltpu.async_copy(tab_sh.at[src_v.at[j]], vals_v.at[b], gsems[b])

        def gwait(b, j):
            pltpu.make_async_copy(
                tab_sh.at[src_v.at[j]], vals_v.at[b], gsems[b]).wait()

        def sfire(b, j):
            pltpu.async_copy(vals_v.at[b], acc_sh.at[dst_v.at[j]], ssems[b],
                             add=True)

        def swait(b, j):
            pltpu.make_async_copy(
                vals_v.at[b], acc_sh.at[dst_v.at[j]], ssems[b]).wait()

        for b in range(nbuf):
            gfire(b, b)

        def body(kk, _):
            j0 = kk * nbuf
            for b in range(nbuf):
                gwait(b, j0 + b)
                sfire(b, j0 + b)
            for b in range(nbuf):
                jn = j0 + b + nbuf
                swait(b, j0 + b)
                @pl.when(jn < n_chunks)
                def _fire(b=b, jn=jn):
                    gfire(b, jn)
            return 0
        lax.fori_loop(0, n_chunks // nbuf, body, 0)
        plsc.subcore_barrier()
        pltpu.sync_copy(acc_sh.at[pl.ds(s * rpt, rpt)],
                        out_hbm.at[pl.ds(c * n_pad + s * rpt, rpt)])

    return k(g2, srcp, dstp)


def _tc_prescale(x_pad, w1, degp):
    np_, dx = x_pad.shape
    h = w1.shape[1]
    blk = 1280
    grid = np_ // blk

    def body(x_ref, w_ref, degp_ref, g1_ref, dinv_ref):
        deg = 1.0 + degp_ref[0] + degp_ref[1]
        dinv = lax.rsqrt(deg)
        hh = jnp.dot(x_ref[...], w_ref[...], preferred_element_type=jnp.float32)
        g1_ref[...] = hh * dinv
        dinv_ref[...] = dinv

    return pl.pallas_call(
        body,
        grid=(grid,),
        in_specs=[
            pl.BlockSpec((blk, dx), lambda i: (i, 0)),
            pl.BlockSpec((dx, h), lambda i: (0, 0)),
            pl.BlockSpec((2, blk, 1), lambda i: (0, i, 0)),
        ],
        out_specs=[
            pl.BlockSpec((blk, h), lambda i: (i, 0)),
            pl.BlockSpec((blk, 1), lambda i: (i, 0)),
        ],
        out_shape=[
            jax.ShapeDtypeStruct((np_, h), jnp.float32),
            jax.ShapeDtypeStruct((np_, 1), jnp.float32),
        ],
    )(x_pad, w1, degp.reshape(2, np_, 1))


def _tc_layer2(agg1p, g1, dinv, b1, w2):
    np_, h = g1.shape
    blk = 1280
    grid = np_ // blk

    def body(aggp_ref, g1_ref, dinv_ref, b1_ref, w2_ref, g2_ref):
        a = (aggp_ref[0] + aggp_ref[1] + g1_ref[...]) * dinv_ref[...]
        z = jnp.maximum(a + b1_ref[...], 0.0)
        h2 = jnp.dot(z, w2_ref[...], preferred_element_type=jnp.float32)
        g2_ref[...] = h2 * dinv_ref[...]

    return pl.pallas_call(
        body,
        grid=(grid,),
        in_specs=[
            pl.BlockSpec((2, blk, h), lambda i: (0, i, 0)),
            pl.BlockSpec((blk, h), lambda i: (i, 0)),
            pl.BlockSpec((blk, 1), lambda i: (i, 0)),
            pl.BlockSpec((1, h), lambda i: (0, 0)),
            pl.BlockSpec((h, 1), lambda i: (0, 0)),
        ],
        out_specs=pl.BlockSpec((blk, 1), lambda i: (i, 0)),
        out_shape=jax.ShapeDtypeStruct((np_, 1), jnp.float32),
    )(agg1p, g1, dinv, b1.reshape(1, h), w2)


def _tc_final(agg2p, g2r, dinvr, b2):
    r, c = g2r.shape

    def body(aggp_ref, g2_ref, dinv_ref, b2_ref, out_ref):
        out_ref[...] = ((aggp_ref[0] + aggp_ref[1] + g2_ref[...])
                        * dinv_ref[...] + b2_ref[...])

    return pl.pallas_call(
        body,
        in_specs=[
            pl.BlockSpec((2, r, c), lambda: (0, 0, 0)),
            pl.BlockSpec((r, c), lambda: (0, 0)),
            pl.BlockSpec((r, c), lambda: (0, 0)),
            pl.BlockSpec((1, 1), lambda: (0, 0)),
        ],
        out_specs=pl.BlockSpec((r, c), lambda: (0, 0)),
        out_shape=jax.ShapeDtypeStruct((r, c), jnp.float32),
    )(agg2p, g2r, dinvr, b2.reshape(1, 1))


def kernel(x, edge_index, W1, b1, W2, b2):
    n, d = x.shape
    e = edge_index.shape[1]
    np_ = ((n + NW * LANES - 1) // (NW * LANES)) * (NW * LANES)
    n_chunks = (e + NW * CH - 1) // (NW * CH)
    n_chunks = ((n_chunks + 3) // 4) * 4
    e_pad = NW * n_chunks * CH
    pad_idx = np_ - 1

    src = edge_index[0]
    dst = edge_index[1]
    fill = jnp.full((e_pad - e,), pad_idx, jnp.int32)
    srcp = jnp.concatenate([src, fill]).reshape(NW, n_chunks, CH)
    dstp = jnp.concatenate([dst, fill]).reshape(NW, n_chunks, CH)
    x_pad = jnp.pad(x, ((0, np_ - n), (0, 0)))

    degp = _sc_degree(dstp, np_, n_chunks)
    g1, dinv = _tc_prescale(x_pad, W1, degp)
    agg1p = _sc_edge_agg(g1, srcp, dstp, np_, n_chunks)
    g2 = _tc_layer2(agg1p.reshape(2, np_, -1), g1, dinv, b1, W2)
    agg2p = _sc_edge_agg_scalar(g2.reshape(np_), srcp, dstp, np_, n_chunks)
    out = _tc_final(agg2p.reshape(2, np_ // 128, 128),
                    g2.reshape(np_ // 128, 128),
                    dinv.reshape(np_ // 128, 128), b2)
    return out.reshape(np_)[:n]


# speedup vs baseline: 2.0299x; 2.0299x over previous
import functools

import jax
import jax.numpy as jnp
from jax import lax
from jax.experimental import pallas as pl
from jax.experimental.pallas import tpu as pltpu
from jax.experimental.pallas import tpu_sc as plsc

NC = 2
NS = 16
NW = NC * NS
CH = 128
LANES = 16


def _sc_mesh():
    return plsc.VectorSubcoreMesh(
        core_axis_name="c", subcore_axis_name="s",
        num_cores=NC, num_subcores=NS)


def _zero_vmem_2d(ref, rows, cols):
    def fill(r, _):
        for k in range(cols // LANES):
            ref[r, pl.ds(k * LANES, LANES)] = jnp.zeros((LANES,), jnp.float32)
        return 0
    lax.fori_loop(0, rows, fill, 0)


def _zero_vmem_1d(ref, n):
    def fill(r, _):
        ref[pl.ds(r * LANES, LANES)] = jnp.zeros((LANES,), jnp.float32)
        return 0
    lax.fori_loop(0, n // LANES, fill, 0)


def _sc_degree(dstp, n_pad, n_chunks):
    rpt = n_pad // NS

    @functools.partial(
        pl.kernel,
        out_type=jax.ShapeDtypeStruct((NC * n_pad,), jnp.float32),
        mesh=_sc_mesh(),
        scratch_types=[
            pltpu.VMEM_SHARED((n_pad,), jnp.float32),
            pltpu.VMEM((n_chunks, CH), jnp.int32),
            pltpu.VMEM((CH,), jnp.float32),
            pltpu.VMEM((rpt,), jnp.float32),
            pltpu.SemaphoreType.DMA,
        ],
    )
    def k(dst_hbm, out_hbm, acc_sh, dst_v, ones_v, buf_v, sem):
        c = lax.axis_index("c")
        s = lax.axis_index("s")
        w = c * NS + s
        pltpu.sync_copy(dst_hbm.at[w], dst_v)

        def fill_ones(r, _):
            ones_v[pl.ds(r * LANES, LANES)] = jnp.ones((LANES,), jnp.float32)
            return 0
        lax.fori_loop(0, CH // LANES, fill_ones, 0)
        _zero_vmem_1d(buf_v, rpt)
        pltpu.sync_copy(buf_v, acc_sh.at[pl.ds(s * rpt, rpt)])
        plsc.subcore_barrier()

        depth = 4

        def fire(j, _):
            @pl.when(j >= depth)
            def _():
                pltpu.make_async_copy(ones_v, acc_sh.at[dst_v.at[j - depth]],
                                      sem).wait()
            pltpu.async_copy(ones_v, acc_sh.at[dst_v.at[j]], sem, add=True)
            return 0
        lax.fori_loop(0, n_chunks, fire, 0)

        def drain(j, _):
            pltpu.make_async_copy(ones_v, acc_sh.at[dst_v.at[j]], sem).wait()
            return 0
        lax.fori_loop(n_chunks - depth, n_chunks, drain, 0)
        plsc.subcore_barrier()
        pltpu.sync_copy(acc_sh.at[pl.ds(s * rpt, rpt)],
                        out_hbm.at[pl.ds(c * n_pad + s * rpt, rpt)])

    return k(dstp)


def _sc_edge_agg(g1s, srcp, dstp, n_pad, n_chunks):
    dh = g1s.shape[1]
    rpt = n_pad // NS
    tch = n_chunks * NC
    hblk = tch // 4
    nbuf = 2

    @functools.partial(
        pl.kernel,
        out_type=jax.ShapeDtypeStruct((NC * n_pad, dh), jnp.float32),
        mesh=_sc_mesh(),
        compiler_params=pltpu.CompilerParams(use_tc_tiling_on_sc=False),
        scratch_types=[
            pltpu.VMEM_SHARED((n_pad, dh), jnp.float32),
            pltpu.VMEM_SHARED((n_pad, dh), jnp.float32),
            pltpu.VMEM((hblk, CH), jnp.int32),
            pltpu.VMEM((hblk, CH), jnp.int32),
            pltpu.VMEM((nbuf, CH, dh), jnp.float32),
            [pltpu.SemaphoreType.DMA] * nbuf,
        ],
    )
    def k(g1s_hbm, src_hbm, dst_hbm, out_hbm, acc_sh, tab_sh, src_v, dst_v,
          rows_v, gsems):
        c = lax.axis_index("c")
        s = lax.axis_index("s")
        pltpu.sync_copy(g1s_hbm.at[pl.ds(c * n_pad + s * rpt, rpt)],
                        tab_sh.at[pl.ds(s * rpt, rpt)])
        _zero_vmem_2d(rows_v.at[0], CH, dh)
        for i in range(rpt // CH):
            pltpu.sync_copy(rows_v.at[0],
                            acc_sh.at[pl.ds(s * rpt + i * CH, CH)])
        plsc.subcore_barrier()

        def gfire(b, j):
            pltpu.async_copy(tab_sh.at[src_v.at[j]], rows_v.at[b], gsems[b])

        def gwait(b, j):
            pltpu.make_async_copy(
                tab_sh.at[src_v.at[j]], rows_v.at[b], gsems[b]).wait()

        def sadd(b, j):
            pltpu.sync_copy(rows_v.at[b], acc_sh.at[dst_v.at[j]], add=True)

        for half in range(tch // hblk):
            pltpu.sync_copy(src_hbm.at[s * (tch // hblk) + half], src_v)
            pltpu.sync_copy(dst_hbm.at[s * (tch // hblk) + half], dst_v)
            gfire(0, 0)

            def body(kk, _):
                j0 = kk * 2
                gwait(0, j0)
                gfire(1, j0 + 1)
                sadd(0, j0)
                gwait(1, j0 + 1)
                @pl.when(j0 + 2 < hblk)
                def _fire():
                    gfire(0, j0 + 2)
                sadd(1, j0 + 1)
                return 0
            lax.fori_loop(0, hblk // 2, body, 0)
        plsc.subcore_barrier()
        pltpu.sync_copy(acc_sh.at[pl.ds(s * rpt, rpt)],
                        out_hbm.at[pl.ds(c * n_pad + s * rpt, rpt)])

    nblk2 = tch // hblk
    return k(g1s, srcp.reshape(NS * nblk2, hblk, CH),
             dstp.reshape(NS * nblk2, hblk, CH))


def _sc_edge_agg_scalar(g2, srcp, dstp, n_pad, n_chunks):
    rpt = n_pad // NS

    nbuf = 2
    assert n_chunks % nbuf == 0

    @functools.partial(
        pl.kernel,
        out_type=jax.ShapeDtypeStruct((NC * n_pad,), jnp.float32),
        mesh=_sc_mesh(),
        scratch_types=[
            pltpu.VMEM_SHARED((n_pad,), jnp.float32),
            pltpu.VMEM_SHARED((n_pad,), jnp.float32),
            pltpu.VMEM((n_chunks, CH), jnp.int32),
            pltpu.VMEM((n_chunks, CH), jnp.int32),
            pltpu.VMEM((nbuf, CH), jnp.float32),
            pltpu.VMEM((rpt,), jnp.float32),
            [pltpu.SemaphoreType.DMA] * nbuf,
            [pltpu.SemaphoreType.DMA] * nbuf,
        ],
    )
    def k(g2_hbm, src_hbm, dst_hbm, out_hbm, acc_sh, tab_sh, src_v, dst_v,
          vals_v, buf_v, gsems, ssems):
        c = lax.axis_index("c")
        s = lax.axis_index("s")
        w = c * NS + s
        pltpu.sync_copy(src_hbm.at[w], src_v)
        pltpu.sync_copy(dst_hbm.at[w], dst_v)
        @pl.when(s == 0)
        def _():
            pltpu.sync_copy(g2_hbm, tab_sh)
        _zero_vmem_1d(buf_v, rpt)
        pltpu.sync_copy(buf_v, acc_sh.at[pl.ds(s * rpt, rpt)])
        plsc.subcore_barrier()

        def gfire(b, j):
            pltpu.async_copy(tab_sh.at[src_v.at[j]], vals_v.at[b], gsems[b])

        def gwait(b, j):
            pltpu.make_async_copy(
                tab_sh.at[src_v.at[j]], vals_v.at[b], gsems[b]).wait()

        def sfire(b, j):
            pltpu.async_copy(vals_v.at[b], acc_sh.at[dst_v.at[j]], ssems[b],
                             add=True)

        def swait(b, j):
            pltpu.make_async_copy(
                vals_v.at[b], acc_sh.at[dst_v.at[j]], ssems[b]).wait()

        for b in range(nbuf):
            gfire(b, b)

        def body(kk, _):
            j0 = kk * nbuf
            for b in range(nbuf):
                gwait(b, j0 + b)
                sfire(b, j0 + b)
            for b in range(nbuf):
                jn = j0 + b + nbuf
                swait(b, j0 + b)
                @pl.when(jn < n_chunks)
                def _fire(b=b, jn=jn):
                    gfire(b, jn)
            return 0
        lax.fori_loop(0, n_chunks // nbuf, body, 0)
        plsc.subcore_barrier()
        pltpu.sync_copy(acc_sh.at[pl.ds(s * rpt, rpt)],
                        out_hbm.at[pl.ds(c * n_pad + s * rpt, rpt)])

    return k(g2, srcp, dstp)


def _tc_prescale(x_pad, w1, degp):
    np_, dx = x_pad.shape
    h = w1.shape[1]
    blk = 1280
    grid = np_ // blk

    hh = h // 2

    def body(x_ref, w_ref, degp_ref, g1s_ref, dinv_ref):
        deg = 1.0 + degp_ref[0] + degp_ref[1]
        dinv = lax.rsqrt(deg)
        hb = jnp.dot(x_ref[...], w_ref[...],
                     preferred_element_type=jnp.float32) * dinv
        g1s_ref[0] = hb[:, :hh]
        g1s_ref[1] = hb[:, hh:]
        dinv_ref[...] = dinv

    return pl.pallas_call(
        body,
        grid=(grid,),
        in_specs=[
            pl.BlockSpec((blk, dx), lambda i: (i, 0)),
            pl.BlockSpec((dx, h), lambda i: (0, 0)),
            pl.BlockSpec((2, blk, 1), lambda i: (0, i, 0)),
        ],
        out_specs=[
            pl.BlockSpec((2, blk, hh), lambda i: (0, i, 0)),
            pl.BlockSpec((blk, 1), lambda i: (i, 0)),
        ],
        out_shape=[
            jax.ShapeDtypeStruct((2, np_, hh), jnp.float32),
            jax.ShapeDtypeStruct((np_, 1), jnp.float32),
        ],
    )(x_pad, w1, degp.reshape(2, np_, 1))


def _tc_layer2(agg1s, g1s, dinv, b1, w2):
    _, np_, hh = g1s.shape
    h = 2 * hh
    blk = 1280
    grid = np_ // blk

    def body(aggs_ref, g1s_ref, dinv_ref, b1_ref, w2_ref, g2_ref):
        dinv = dinv_ref[...]
        w2 = w2_ref[...]
        h2 = jnp.zeros((blk, 1), jnp.float32)
        for hf in range(2):
            a = (aggs_ref[hf] + g1s_ref[hf]) * dinv
            z = jnp.maximum(a + b1_ref[hf][None, :], 0.0)
            h2 = h2 + jnp.dot(z, w2[hf * hh:(hf + 1) * hh],
                              preferred_element_type=jnp.float32)
        g2_ref[...] = h2 * dinv

    return pl.pallas_call(
        body,
        grid=(grid,),
        in_specs=[
            pl.BlockSpec((2, blk, hh), lambda i: (0, i, 0)),
            pl.BlockSpec((2, blk, hh), lambda i: (0, i, 0)),
            pl.BlockSpec((blk, 1), lambda i: (i, 0)),
            pl.BlockSpec((2, hh), lambda i: (0, 0)),
            pl.BlockSpec((h, 1), lambda i: (0, 0)),
        ],
        out_specs=pl.BlockSpec((blk, 1), lambda i: (i, 0)),
        out_shape=jax.ShapeDtypeStruct((np_, 1), jnp.float32),
    )(agg1s, g1s, dinv, b1.reshape(2, hh), w2)


def _tc_final(agg2p, g2r, dinvr, b2):
    r, c = g2r.shape

    def body(aggp_ref, g2_ref, dinv_ref, b2_ref, out_ref):
        out_ref[...] = ((aggp_ref[0] + aggp_ref[1] + g2_ref[...])
                        * dinv_ref[...] + b2_ref[...])

    return pl.pallas_call(
        body,
        in_specs=[
            pl.BlockSpec((2, r, c), lambda: (0, 0, 0)),
            pl.BlockSpec((r, c), lambda: (0, 0)),
            pl.BlockSpec((r, c), lambda: (0, 0)),
            pl.BlockSpec((1, 1), lambda: (0, 0)),
        ],
        out_specs=pl.BlockSpec((r, c), lambda: (0, 0)),
        out_shape=jax.ShapeDtypeStruct((r, c), jnp.float32),
    )(agg2p, g2r, dinvr, b2.reshape(1, 1))


def kernel(x, edge_index, W1, b1, W2, b2):
    n, d = x.shape
    e = edge_index.shape[1]
    np_ = ((n + NW * LANES - 1) // (NW * LANES)) * (NW * LANES)
    n_chunks = (e + NW * CH - 1) // (NW * CH)
    n_chunks = ((n_chunks + 3) // 4) * 4
    e_pad = NW * n_chunks * CH
    pad_idx = np_ - 1

    src = edge_index[0]
    dst = edge_index[1]
    fill = jnp.full((e_pad - e,), pad_idx, jnp.int32)
    srcp = jnp.concatenate([src, fill]).reshape(NW, n_chunks, CH)
    dstp = jnp.concatenate([dst, fill]).reshape(NW, n_chunks, CH)
    x_pad = jnp.pad(x, ((0, np_ - n), (0, 0)))

    degp = _sc_degree(dstp, np_, n_chunks)
    g1s, dinv = _tc_prescale(x_pad, W1, degp)
    agg1s = _sc_edge_agg(g1s.reshape(2 * np_, -1), srcp, dstp, np_,
                         n_chunks)
    g2 = _tc_layer2(agg1s.reshape(2, np_, -1), g1s, dinv, b1, W2)
    agg2p = _sc_edge_agg_scalar(g2.reshape(np_), srcp, dstp, np_, n_chunks)
    out = _tc_final(agg2p.reshape(2, np_ // 128, 128),
                    g2.reshape(np_ // 128, 128),
                    dinv.reshape(np_ // 128, 128), b2)
    return out.reshape(np_)[:n]
